# baseline (speedup 1.0000x reference)
import functools

import jax
import jax.numpy as jnp
from jax import lax
from jax.experimental import pallas as pl
from jax.experimental.pallas import tpu as pltpu
from jax.experimental.pallas import tpu_sc as plsc

N = 10000
E = 320000
D_IN = 128
D_HID = 128
D_CLS = 40
D_CLS_PAD = 128

NC = 2
NS = 16
NW = NC * NS
L = 16

N_PAD = 10240
RPS = N_PAD // NS
C = 128
TRIPS = 80
CHUNKS = NW * TRIPS
E_PAD = CHUNKS * C
NB = 2


def _sc_mesh():
  return plsc.VectorSubcoreMesh(
      core_axis_name="c", subcore_axis_name="s", num_cores=NC, num_subcores=NS
  )


HR = N_PAD // 128


@functools.partial(
    pl.kernel,
    out_type=jax.ShapeDtypeStruct((NC, HR, 128), jnp.float32),
    mesh=_sc_mesh(),
    compiler_params=pltpu.CompilerParams(needs_layout_passes=False),
    scratch_types=[
        pltpu.VMEM((TRIPS, C), jnp.int32),
        pltpu.VMEM((HR, 128), jnp.float32),
        pltpu.VMEM((HR,), jnp.int32),
        pltpu.VMEM_SHARED((HR, 128), jnp.float32),
    ],
)
def _sc_degree(dst_hbm, out_hbm, didx_v, hist_v, iota_v, acc_sh):
  cid = lax.axis_index("c")
  sid = lax.axis_index("s")
  wid = sid * NC + cid

  pltpu.sync_copy(dst_hbm.at[pl.ds(wid * TRIPS, TRIPS)], didx_v)

  def zbody(r, carry):
    for j in range(8):
      hist_v[r, pl.ds(j * L, L)] = jnp.zeros((L,), jnp.float32)
    return carry
  lax.fori_loop(0, HR, zbody, 0)
  for j in range(HR // L):
    iota_v[pl.ds(j * L, L)] = lax.iota(jnp.int32, L) + j * L

  @pl.when(sid == 0)
  def _():
    pltpu.sync_copy(hist_v, acc_sh)

  ones = jnp.ones((L,), jnp.float32)

  def body(i, carry):
    for k in range(C // L):
      idx = didx_v[i, pl.ds(k * L, L)]
      plsc.addupdate_scatter(hist_v, [idx >> 7, idx & 127], ones)
    return carry

  lax.fori_loop(0, TRIPS, body, 0)
  plsc.subcore_barrier()

  pltpu.sync_copy(hist_v, acc_sh.at[iota_v], add=True)
  plsc.subcore_barrier()
  @pl.when(sid == 0)
  def _():
    pltpu.sync_copy(acc_sh, out_hbm.at[cid])


BLK = 40
NBLK = TRIPS // BLK


def _make_sc_aggregate(D):
  @functools.partial(
      pl.kernel,
      out_type=jax.ShapeDtypeStruct((NC, N_PAD, D), jnp.float32),
      mesh=_sc_mesh(),
      scratch_types=[
          pltpu.VMEM((BLK, C), jnp.int32),
          pltpu.VMEM((BLK, C), jnp.int32),
          pltpu.VMEM_SHARED((N_PAD, D), jnp.float32),
      ]
      + [pltpu.VMEM((C, D), jnp.float32) for _ in range(NB)]
      + [pltpu.SemaphoreType.DMA for _ in range(NB)],
  )
  def _sc_aggregate(table_hbm, src_hbm, dst_hbm, out_hbm, sidx_v, didx_v,
                    acc_sh, *bufs_and_sems):
    rows = bufs_and_sems[:NB]
    sems = bufs_and_sems[NB:]
    cid = lax.axis_index("c")
    sid = lax.axis_index("s")
    wid = sid * NC + cid

    def zbody(r, carry):
      for j in range(D // L):
        rows[0][r, pl.ds(j * L, L)] = jnp.zeros((L,), jnp.float32)
      return carry
    lax.fori_loop(0, C, zbody, 0)
    for j in range(RPS // C):
      pltpu.sync_copy(rows[0], acc_sh.at[pl.ds(sid * RPS + j * C, C)])
    plsc.subcore_barrier()

    def gather(k, b):
      pltpu.async_copy(table_hbm.at[sidx_v.at[k]], rows[b], sems[b])

    for blk in range(NBLK):
      base = wid * TRIPS + blk * BLK
      pltpu.sync_copy(src_hbm.at[pl.ds(base, BLK)], sidx_v)
      pltpu.sync_copy(dst_hbm.at[pl.ds(base, BLK)], didx_v)
      gather(0, 0)

      def body(o, carry):
        for b in range(NB):
          k = o * NB + b
          pltpu.make_async_copy(table_hbm.at[sidx_v.at[k]], rows[b],
                                sems[b]).wait()
          @pl.when(k < BLK - 1)
          def _():
            gather(k + 1, (b + 1) % NB)
          pltpu.sync_copy(rows[b], acc_sh.at[didx_v.at[k]], add=True)
        return carry

      lax.fori_loop(0, BLK // NB, body, 0)

    plsc.subcore_barrier()
    pltpu.sync_copy(
        acc_sh.at[pl.ds(sid * RPS, RPS)],
        out_hbm.at[cid, pl.ds(sid * RPS, RPS)],
    )

  return _sc_aggregate


_sc_aggregate_128 = _make_sc_aggregate(D_HID)


_R = 512
_GRID = N_PAD // _R


def _tc_layer0(feat, w0, d0, d1):
  def body(x_ref, w_ref, d0_ref, d1_ref, t_ref, n_ref):
    deg = d0_ref[...] + d1_ref[...]
    norm = lax.rsqrt(jnp.maximum(deg, 1.0))
    n_ref[...] = norm
    y = jnp.dot(x_ref[...], w_ref[...], preferred_element_type=jnp.float32)
    t_ref[...] = y * norm

  rows = 400
  return pl.pallas_call(
      body,
      grid=(N // rows,),
      in_specs=[
          pl.BlockSpec((rows, D_IN), lambda i: (i, 0)),
          pl.BlockSpec((D_IN, D_HID), lambda i: (0, 0)),
          pl.BlockSpec((rows, 1), lambda i: (i, 0)),
          pl.BlockSpec((rows, 1), lambda i: (i, 0)),
      ],
      out_specs=[
          pl.BlockSpec((rows, D_HID), lambda i: (i, 0)),
          pl.BlockSpec((rows, 1), lambda i: (i, 0)),
      ],
      out_shape=[
          jax.ShapeDtypeStruct((N_PAD, D_HID), jnp.float32),
          jax.ShapeDtypeStruct((N_PAD, 1), jnp.float32),
      ],
  )(feat, w0, d0, d1)


def _tc_mid(p0, p1, norm, b, w, d_out):
  def body(p0_ref, p1_ref, n_ref, b_ref, w_ref, o_ref):
    nrm = n_ref[...]
    h = (p0_ref[...] + p1_ref[...]) * nrm + b_ref[...]
    h = jnp.maximum(h, 0.0)
    o_ref[...] = (
        jnp.dot(h, w_ref[...], preferred_element_type=jnp.float32) * nrm
    )

  d_in = p0.shape[-1]
  return pl.pallas_call(
      body,
      grid=(_GRID,),
      in_specs=[
          pl.BlockSpec((_R, d_in), lambda i: (i, 0)),
          pl.BlockSpec((_R, d_in), lambda i: (i, 0)),
          pl.BlockSpec((_R, 1), lambda i: (i, 0)),
          pl.BlockSpec((1, d_in), lambda i: (0, 0)),
          pl.BlockSpec((d_in, d_out), lambda i: (0, 0)),
      ],
      out_specs=pl.BlockSpec((_R, d_out), lambda i: (i, 0)),
      out_shape=jax.ShapeDtypeStruct((N_PAD, d_out), jnp.float32),
  )(p0, p1, norm, b, w)


def _tc_final(p0, p1, norm, b):
  rows = 400

  def body(p0_ref, p1_ref, n_ref, b_ref, o_ref):
    y = (p0_ref[...] + p1_ref[...]) * n_ref[...] + b_ref[...]
    o_ref[...] = y[:, :D_CLS]

  return pl.pallas_call(
      body,
      grid=(N // rows,),
      in_specs=[
          pl.BlockSpec((rows, D_CLS_PAD), lambda i: (i, 0)),
          pl.BlockSpec((rows, D_CLS_PAD), lambda i: (i, 0)),
          pl.BlockSpec((rows, 1), lambda i: (i, 0)),
          pl.BlockSpec((1, D_CLS_PAD), lambda i: (0, 0)),
      ],
      out_specs=pl.BlockSpec((rows, D_CLS), lambda i: (i, 0)),
      out_shape=jax.ShapeDtypeStruct((N, D_CLS), jnp.float32),
  )(p0, p1, norm, b)


def kernel(features, edge_index, W0, b0, W1, b1, W2, b2):
  src = edge_index[0]
  dst = edge_index[1]

  npad = E_PAD - E
  pad_ar = jnp.arange(npad, dtype=jnp.int32)
  src2 = jnp.concatenate([src, pad_ar % N]).reshape(CHUNKS, C)
  dst2 = jnp.concatenate([dst, N + (pad_ar % (N_PAD - N))]).reshape(CHUNKS, C)
  src2 = src2.reshape(TRIPS, NW, C).transpose(1, 0, 2).reshape(CHUNKS, C)
  dst2 = dst2.reshape(TRIPS, NW, C).transpose(1, 0, 2).reshape(CHUNKS, C)

  feat = features
  w2p = jnp.pad(W2, ((0, 0), (0, D_CLS_PAD - D_CLS)))
  b2p = jnp.pad(b2, (0, D_CLS_PAD - D_CLS)).reshape(1, D_CLS_PAD)
  b0r = b0.reshape(1, D_HID)
  b1r = b1.reshape(1, D_HID)

  deg = _sc_degree(dst2).reshape(NC, N_PAD, 1)
  t0, norm = _tc_layer0(feat, W0, deg[0], deg[1])

  a0 = _sc_aggregate_128(t0, src2, dst2)
  t1 = _tc_mid(a0[0], a0[1], norm, b0r, W1, D_HID)

  a1 = _sc_aggregate_128(t1, src2, dst2)
  t2 = _tc_mid(a1[0], a1[1], norm, b1r, w2p, D_CLS_PAD)

  a2 = _sc_aggregate_128(t2, src2, dst2)
  return _tc_final(a2[0], a2[1], norm, b2p)


# speedup vs baseline: 1.0285x; 1.0285x over previous
import functools

import jax
import jax.numpy as jnp
from jax import lax
from jax.experimental import pallas as pl
from jax.experimental.pallas import tpu as pltpu
from jax.experimental.pallas import tpu_sc as plsc

N = 10000
E = 320000
D_IN = 128
D_HID = 128
D_CLS = 40
D_CLS_PAD = 128

NC = 2
NS = 16
NW = NC * NS
L = 16

N_PAD = 10240
RPS = N_PAD // NS
C = 128
TRIPS = 80
CHUNKS = NW * TRIPS
E_PAD = CHUNKS * C
NB = 2


def _sc_mesh():
  return plsc.VectorSubcoreMesh(
      core_axis_name="c", subcore_axis_name="s", num_cores=NC, num_subcores=NS
  )


HR = N_PAD // 128


@functools.partial(
    pl.kernel,
    out_type=jax.ShapeDtypeStruct((NC, HR, 128), jnp.float32),
    mesh=_sc_mesh(),
    compiler_params=pltpu.CompilerParams(needs_layout_passes=False),
    scratch_types=[
        pltpu.VMEM((TRIPS, C), jnp.int32),
        pltpu.VMEM((HR, 128), jnp.float32),
        pltpu.VMEM((HR,), jnp.int32),
        pltpu.VMEM_SHARED((HR, 128), jnp.float32),
    ],
)
def _sc_degree(dst_hbm, out_hbm, didx_v, hist_v, iota_v, acc_sh):
  cid = lax.axis_index("c")
  sid = lax.axis_index("s")
  wid = sid * NC + cid

  pltpu.sync_copy(dst_hbm.at[pl.ds(wid * TRIPS, TRIPS)], didx_v)

  def zbody(r, carry):
    for j in range(8):
      hist_v[r, pl.ds(j * L, L)] = jnp.zeros((L,), jnp.float32)
    return carry
  lax.fori_loop(0, HR, zbody, 0)
  for j in range(HR // L):
    iota_v[pl.ds(j * L, L)] = lax.iota(jnp.int32, L) + j * L

  @pl.when(sid == 0)
  def _():
    pltpu.sync_copy(hist_v, acc_sh)

  ones = jnp.ones((L,), jnp.float32)

  def body(i, carry):
    for k in range(C // L):
      idx = didx_v[i, pl.ds(k * L, L)]
      plsc.addupdate_scatter(hist_v, [idx >> 7, idx & 127], ones)
    return carry

  lax.fori_loop(0, TRIPS, body, 0)
  plsc.subcore_barrier()

  pltpu.sync_copy(hist_v, acc_sh.at[iota_v], add=True)
  plsc.subcore_barrier()
  @pl.when(sid == 0)
  def _():
    pltpu.sync_copy(acc_sh, out_hbm.at[cid])


BLK = 40
NBLK = TRIPS // BLK


def _make_sc_aggregate(D):
  @functools.partial(
      pl.kernel,
      out_type=jax.ShapeDtypeStruct((NC, N_PAD, D), jnp.float32),
      mesh=_sc_mesh(),
      scratch_types=[
          pltpu.VMEM((BLK, C), jnp.int32),
          pltpu.VMEM((BLK, C), jnp.int32),
          pltpu.VMEM_SHARED((N_PAD, D), jnp.float32),
      ]
      + [pltpu.VMEM((C, D), jnp.float32) for _ in range(NB)]
      + [pltpu.SemaphoreType.DMA for _ in range(NB)],
  )
  def _sc_aggregate(table_hbm, src_hbm, dst_hbm, out_hbm, sidx_v, didx_v,
                    acc_sh, *bufs_and_sems):
    rows = bufs_and_sems[:NB]
    sems = bufs_and_sems[NB:]
    cid = lax.axis_index("c")
    sid = lax.axis_index("s")
    wid = sid * NC + cid

    def zbody(r, carry):
      for j in range(D // L):
        rows[0][r, pl.ds(j * L, L)] = jnp.zeros((L,), jnp.float32)
      return carry
    lax.fori_loop(0, C, zbody, 0)
    for j in range(RPS // C):
      pltpu.sync_copy(rows[0], acc_sh.at[pl.ds(sid * RPS + j * C, C)])
    plsc.subcore_barrier()

    def gather(k, b):
      pltpu.async_copy(table_hbm.at[sidx_v.at[k]], rows[b], sems[b])

    for blk in range(NBLK):
      base = wid * TRIPS + blk * BLK
      pltpu.sync_copy(src_hbm.at[pl.ds(base, BLK)], sidx_v)
      pltpu.sync_copy(dst_hbm.at[pl.ds(base, BLK)], didx_v)
      gather(0, 0)

      def body(o, carry):
        for b in range(NB):
          k = o * NB + b
          pltpu.make_async_copy(table_hbm.at[sidx_v.at[k]], rows[b],
                                sems[b]).wait()
          @pl.when(k < BLK - 1)
          def _():
            gather(k + 1, (b + 1) % NB)
          pltpu.sync_copy(rows[b], acc_sh.at[didx_v.at[k]], add=True)
        return carry

      lax.fori_loop(0, BLK // NB, body, 0)

    plsc.subcore_barrier()
    pltpu.sync_copy(
        acc_sh.at[pl.ds(sid * RPS, RPS)],
        out_hbm.at[cid, pl.ds(sid * RPS, RPS)],
    )

  return _sc_aggregate


_sc_aggregate_128 = _make_sc_aggregate(D_HID)


_R = 512
_GRID = N_PAD // _R


def _tc_layer0(feat, w0, degp):
  def body(x_ref, w_ref, d_ref, t_ref, n_ref):
    deg = d_ref[0] + d_ref[1]
    norm = lax.rsqrt(jnp.maximum(deg, 1.0))
    n_ref[...] = norm
    y = jnp.dot(x_ref[...], w_ref[...], preferred_element_type=jnp.float32)
    t_ref[...] = y * norm

  rows = 400
  return pl.pallas_call(
      body,
      grid=(N // rows,),
      in_specs=[
          pl.BlockSpec((rows, D_IN), lambda i: (i, 0)),
          pl.BlockSpec((D_IN, D_HID), lambda i: (0, 0)),
          pl.BlockSpec((NC, rows, 1), lambda i: (0, i, 0)),
      ],
      out_specs=[
          pl.BlockSpec((rows, D_HID), lambda i: (i, 0)),
          pl.BlockSpec((rows, 1), lambda i: (i, 0)),
      ],
      out_shape=[
          jax.ShapeDtypeStruct((N_PAD, D_HID), jnp.float32),
          jax.ShapeDtypeStruct((N_PAD, 1), jnp.float32),
      ],
  )(feat, w0, degp)


def _tc_mid(parts, norm, b, w, d_out):
  def body(p_ref, n_ref, b_ref, w_ref, o_ref):
    nrm = n_ref[...]
    h = (p_ref[0] + p_ref[1]) * nrm + b_ref[...]
    h = jnp.maximum(h, 0.0)
    o_ref[...] = (
        jnp.dot(h, w_ref[...], preferred_element_type=jnp.float32) * nrm
    )

  d_in = parts.shape[-1]
  return pl.pallas_call(
      body,
      grid=(_GRID,),
      in_specs=[
          pl.BlockSpec((NC, _R, d_in), lambda i: (0, i, 0)),
          pl.BlockSpec((_R, 1), lambda i: (i, 0)),
          pl.BlockSpec((1, d_in), lambda i: (0, 0)),
          pl.BlockSpec((d_in, d_out), lambda i: (0, 0)),
      ],
      out_specs=pl.BlockSpec((_R, d_out), lambda i: (i, 0)),
      out_shape=jax.ShapeDtypeStruct((N_PAD, d_out), jnp.float32),
  )(parts, norm, b, w)


def _tc_final(parts, norm, b):
  rows = 400

  def body(p_ref, n_ref, b_ref, o_ref):
    y = (p_ref[0] + p_ref[1]) * n_ref[...] + b_ref[...]
    o_ref[...] = y[:, :D_CLS]

  return pl.pallas_call(
      body,
      grid=(N // rows,),
      in_specs=[
          pl.BlockSpec((NC, rows, D_CLS_PAD), lambda i: (0, i, 0)),
          pl.BlockSpec((rows, 1), lambda i: (i, 0)),
          pl.BlockSpec((1, D_CLS_PAD), lambda i: (0, 0)),
      ],
      out_specs=pl.BlockSpec((rows, D_CLS), lambda i: (i, 0)),
      out_shape=jax.ShapeDtypeStruct((N, D_CLS), jnp.float32),
  )(parts, norm, b)


def kernel(features, edge_index, W0, b0, W1, b1, W2, b2):
  src = edge_index[0]
  dst = edge_index[1]

  npad = E_PAD - E
  pad_ar = jnp.arange(npad, dtype=jnp.int32)
  src2 = jnp.concatenate([src, pad_ar % N]).reshape(CHUNKS, C)
  dst2 = jnp.concatenate([dst, N + (pad_ar % (N_PAD - N))]).reshape(CHUNKS, C)
  src2 = src2.reshape(TRIPS, NW, C).transpose(1, 0, 2).reshape(CHUNKS, C)
  dst2 = dst2.reshape(TRIPS, NW, C).transpose(1, 0, 2).reshape(CHUNKS, C)

  feat = features
  w2p = jnp.pad(W2, ((0, 0), (0, D_CLS_PAD - D_CLS)))
  b2p = jnp.pad(b2, (0, D_CLS_PAD - D_CLS)).reshape(1, D_CLS_PAD)
  b0r = b0.reshape(1, D_HID)
  b1r = b1.reshape(1, D_HID)

  deg = _sc_degree(dst2).reshape(NC, N_PAD, 1)
  t0, norm = _tc_layer0(feat, W0, deg)

  a0 = _sc_aggregate_128(t0, src2, dst2)
  t1 = _tc_mid(a0, norm, b0r, W1, D_HID)

  a1 = _sc_aggregate_128(t1, src2, dst2)
  t2 = _tc_mid(a1, norm, b1r, w2p, D_CLS_PAD)

  a2 = _sc_aggregate_128(t2, src2, dst2)
  return _tc_final(a2, norm, b2p)


# speedup vs baseline: 1.0535x; 1.0243x over previous
import functools

import jax
import jax.numpy as jnp
from jax import lax
from jax.experimental import pallas as pl
from jax.experimental.pallas import tpu as pltpu
from jax.experimental.pallas import tpu_sc as plsc

N = 10000
E = 320000
D_IN = 128
D_HID = 128
D_CLS = 40
D_CLS_PAD = 128

NC = 2
NS = 16
NW = NC * NS
L = 16

N_PAD = 10240
RPS = N_PAD // NS
C = 128
TRIPS = 80
CHUNKS = NW * TRIPS
E_PAD = CHUNKS * C
NB = 2


def _sc_mesh():
  return plsc.VectorSubcoreMesh(
      core_axis_name="c", subcore_axis_name="s", num_cores=NC, num_subcores=NS
  )


HR = N_PAD // 128


@functools.partial(
    pl.kernel,
    out_type=jax.ShapeDtypeStruct((NC, HR, 128), jnp.float32),
    mesh=_sc_mesh(),
    compiler_params=pltpu.CompilerParams(needs_layout_passes=False),
    scratch_types=[
        pltpu.VMEM((TRIPS, C), jnp.int32),
        pltpu.VMEM((HR, 128), jnp.float32),
        pltpu.VMEM((HR,), jnp.int32),
        pltpu.VMEM_SHARED((HR, 128), jnp.float32),
    ],
)
def _sc_degree(dst_hbm, out_hbm, didx_v, hist_v, iota_v, acc_sh):
  cid = lax.axis_index("c")
  sid = lax.axis_index("s")
  wid = sid * NC + cid

  pltpu.sync_copy(dst_hbm.at[pl.ds(wid * TRIPS, TRIPS)], didx_v)

  def zbody(r, carry):
    for j in range(8):
      hist_v[r, pl.ds(j * L, L)] = jnp.zeros((L,), jnp.float32)
    return carry
  lax.fori_loop(0, HR, zbody, 0)
  for j in range(HR // L):
    iota_v[pl.ds(j * L, L)] = lax.iota(jnp.int32, L) + j * L

  @pl.when(sid == 0)
  def _():
    pltpu.sync_copy(hist_v, acc_sh)

  ones = jnp.ones((L,), jnp.float32)

  def body(i, carry):
    for k in range(C // L):
      idx = didx_v[i, pl.ds(k * L, L)]
      plsc.addupdate_scatter(hist_v, [idx >> 7, idx & 127], ones)
    return carry

  lax.fori_loop(0, TRIPS, body, 0)
  plsc.subcore_barrier()

  pltpu.sync_copy(hist_v, acc_sh.at[iota_v], add=True)
  plsc.subcore_barrier()
  @pl.when(sid == 0)
  def _():
    pltpu.sync_copy(acc_sh, out_hbm.at[cid])


BLK = 40
NBLK = TRIPS // BLK


def _make_sc_aggregate(D):
  @functools.partial(
      pl.kernel,
      out_type=jax.ShapeDtypeStruct((NC, N_PAD, D), jnp.float32),
      mesh=_sc_mesh(),
      scratch_types=[
          pltpu.VMEM((BLK, C), jnp.int32),
          pltpu.VMEM((BLK, C), jnp.int32),
          pltpu.VMEM_SHARED((N_PAD, D), jnp.float32),
      ]
      + [pltpu.VMEM((C, D), jnp.float32) for _ in range(NB)]
      + [pltpu.SemaphoreType.DMA for _ in range(2 * NB)],
  )
  def _sc_aggregate(table_hbm, src_hbm, dst_hbm, out_hbm, sidx_v, didx_v,
                    acc_sh, *bufs_and_sems):
    rows = bufs_and_sems[:NB]
    sems = bufs_and_sems[NB:]
    cid = lax.axis_index("c")
    sid = lax.axis_index("s")
    wid = sid * NC + cid

    def zbody(r, carry):
      for j in range(D // L):
        rows[0][r, pl.ds(j * L, L)] = jnp.zeros((L,), jnp.float32)
      return carry
    lax.fori_loop(0, C, zbody, 0)
    for j in range(RPS // C):
      pltpu.sync_copy(rows[0], acc_sh.at[pl.ds(sid * RPS + j * C, C)])
    plsc.subcore_barrier()

    H = C // 2

    def gather(k, b):
      for q in range(2):
        pltpu.async_copy(table_hbm.at[sidx_v.at[k, pl.ds(q * H, H)]],
                         rows[b].at[pl.ds(q * H, H)], sems[2 * b + q])

    for blk in range(NBLK):
      base = wid * TRIPS + blk * BLK
      pltpu.sync_copy(src_hbm.at[pl.ds(base, BLK)], sidx_v)
      pltpu.sync_copy(dst_hbm.at[pl.ds(base, BLK)], didx_v)
      gather(0, 0)

      def body(o, carry):
        for b in range(NB):
          k = o * NB + b
          for q in range(2):
            pltpu.make_async_copy(table_hbm.at[sidx_v.at[k, pl.ds(q * H, H)]],
                                  rows[b].at[pl.ds(q * H, H)],
                                  sems[2 * b + q]).wait()
          @pl.when(k < BLK - 1)
          def _():
            gather(k + 1, (b + 1) % NB)
          pltpu.sync_copy(rows[b], acc_sh.at[didx_v.at[k]], add=True)
        return carry

      lax.fori_loop(0, BLK // NB, body, 0)

    plsc.subcore_barrier()
    pltpu.sync_copy(
        acc_sh.at[pl.ds(sid * RPS, RPS)],
        out_hbm.at[cid, pl.ds(sid * RPS, RPS)],
    )

  return _sc_aggregate


_sc_aggregate_128 = _make_sc_aggregate(D_HID)


_R = 512
_GRID = N_PAD // _R


def _tc_layer0(feat, w0, degp):
  def body(x_ref, w_ref, d_ref, t_ref, n_ref):
    deg = d_ref[0] + d_ref[1]
    norm = lax.rsqrt(jnp.maximum(deg, 1.0))
    n_ref[...] = norm
    y = jnp.dot(x_ref[...], w_ref[...], preferred_element_type=jnp.float32)
    t_ref[...] = y * norm

  rows = 400
  return pl.pallas_call(
      body,
      grid=(N // rows,),
      in_specs=[
          pl.BlockSpec((rows, D_IN), lambda i: (i, 0)),
          pl.BlockSpec((D_IN, D_HID), lambda i: (0, 0)),
          pl.BlockSpec((NC, rows, 1), lambda i: (0, i, 0)),
      ],
      out_specs=[
          pl.BlockSpec((rows, D_HID), lambda i: (i, 0)),
          pl.BlockSpec((rows, 1), lambda i: (i, 0)),
      ],
      out_shape=[
          jax.ShapeDtypeStruct((N_PAD, D_HID), jnp.float32),
          jax.ShapeDtypeStruct((N_PAD, 1), jnp.float32),
      ],
  )(feat, w0, degp)


def _tc_mid(parts, norm, b, w, d_out):
  def body(p_ref, n_ref, b_ref, w_ref, o_ref):
    nrm = n_ref[...]
    h = (p_ref[0] + p_ref[1]) * nrm + b_ref[...]
    h = jnp.maximum(h, 0.0)
    o_ref[...] = (
        jnp.dot(h, w_ref[...], preferred_element_type=jnp.float32) * nrm
    )

  d_in = parts.shape[-1]
  return pl.pallas_call(
      body,
      grid=(_GRID,),
      in_specs=[
          pl.BlockSpec((NC, _R, d_in), lambda i: (0, i, 0)),
          pl.BlockSpec((_R, 1), lambda i: (i, 0)),
          pl.BlockSpec((1, d_in), lambda i: (0, 0)),
          pl.BlockSpec((d_in, d_out), lambda i: (0, 0)),
      ],
      out_specs=pl.BlockSpec((_R, d_out), lambda i: (i, 0)),
      out_shape=jax.ShapeDtypeStruct((N_PAD, d_out), jnp.float32),
  )(parts, norm, b, w)


def _tc_final(parts, norm, b):
  rows = 400

  def body(p_ref, n_ref, b_ref, o_ref):
    y = (p_ref[0] + p_ref[1]) * n_ref[...] + b_ref[...]
    o_ref[...] = y[:, :D_CLS]

  return pl.pallas_call(
      body,
      grid=(N // rows,),
      in_specs=[
          pl.BlockSpec((NC, rows, D_CLS_PAD), lambda i: (0, i, 0)),
          pl.BlockSpec((rows, 1), lambda i: (i, 0)),
          pl.BlockSpec((1, D_CLS_PAD), lambda i: (0, 0)),
      ],
      out_specs=pl.BlockSpec((rows, D_CLS), lambda i: (i, 0)),
      out_shape=jax.ShapeDtypeStruct((N, D_CLS), jnp.float32),
  )(parts, norm, b)


def kernel(features, edge_index, W0, b0, W1, b1, W2, b2):
  src = edge_index[0]
  dst = edge_index[1]

  npad = E_PAD - E
  pad_ar = jnp.arange(npad, dtype=jnp.int32)
  src2 = jnp.concatenate([src, pad_ar % N]).reshape(CHUNKS, C)
  dst2 = jnp.concatenate([dst, N + (pad_ar % (N_PAD - N))]).reshape(CHUNKS, C)
  src2 = src2.reshape(TRIPS, NW, C).transpose(1, 0, 2).reshape(CHUNKS, C)
  dst2 = dst2.reshape(TRIPS, NW, C).transpose(1, 0, 2).reshape(CHUNKS, C)

  feat = features
  w2p = jnp.pad(W2, ((0, 0), (0, D_CLS_PAD - D_CLS)))
  b2p = jnp.pad(b2, (0, D_CLS_PAD - D_CLS)).reshape(1, D_CLS_PAD)
  b0r = b0.reshape(1, D_HID)
  b1r = b1.reshape(1, D_HID)

  deg = _sc_degree(dst2).reshape(NC, N_PAD, 1)
  t0, norm = _tc_layer0(feat, W0, deg)

  a0 = _sc_aggregate_128(t0, src2, dst2)
  t1 = _tc_mid(a0, norm, b0r, W1, D_HID)

  a1 = _sc_aggregate_128(t1, src2, dst2)
  t2 = _tc_mid(a1, norm, b1r, w2p, D_CLS_PAD)

  a2 = _sc_aggregate_128(t2, src2, dst2)
  return _tc_final(a2, norm, b2p)
